# Initial kernel scaffold; baseline (speedup 1.0000x reference)
import functools

import jax
import jax.numpy as jnp
from jax import lax
from jax.experimental import pallas as pl
from jax.experimental.pallas import tpu as pltpu
from jax.experimental.pallas import tpu_sc as plsc

NC = 2
NS = 16
L = 16


def _mm_body(x_ref, w_ref, o_ref):
    o_ref[...] = jnp.dot(x_ref[...], w_ref[...],
                         preferred_element_type=jnp.float32)


def _tc_matmul(xf, W):
    M, D = xf.shape
    OUT = W.shape[1]
    BLK = 2500
    return pl.pallas_call(
        _mm_body,
        grid=(M // BLK,),
        in_specs=[pl.BlockSpec((BLK, D), lambda i: (i, 0)),
                  pl.BlockSpec((D, OUT), lambda i: (0, 0))],
        out_specs=pl.BlockSpec((BLK, OUT), lambda i: (i, 0)),
        out_shape=jax.ShapeDtypeStruct((M, OUT), jnp.float32),
    )(xf, W)


def _rsqrt16(x):
    xi = plsc.bitcast(x, jnp.int32)
    yi = jnp.int32(0x5F3759DF) - lax.shift_right_logical(xi, 1)
    y = plsc.bitcast(yi, jnp.float32)
    hx = x * 0.5
    for _ in range(3):
        y = y * (1.5 - hx * y * y)
    return y


@functools.lru_cache(maxsize=None)
def _make_sc_conv(B, N, OUT, E2p):
    NP = ((N + NS * L - 1) // (NS * L)) * (NS * L)
    NPT = NP // NS
    ET = E2p // NS
    K = 128
    C = ET // K
    RB = B // NC
    n_full = N // NPT
    n_rem = N - n_full * NPT

    mesh = plsc.VectorSubcoreMesh(core_axis_name="c", subcore_axis_name="s")

    @functools.partial(
        pl.kernel,
        out_type=jax.ShapeDtypeStruct((B, N, OUT), jnp.float32),
        mesh=mesh,
        scratch_types=[
            pltpu.VMEM_SHARED((NP, OUT), jnp.float32),
            pltpu.VMEM_SHARED((NS, NP), jnp.float32),
            pltpu.VMEM_SHARED((NP,), jnp.float32),
            pltpu.VMEM((NP,), jnp.float32),
            pltpu.VMEM((NP,), jnp.float32),
            pltpu.VMEM((ET,), jnp.int32),
            pltpu.VMEM((ET,), jnp.int32),
            pltpu.VMEM((ET,), jnp.float32),
            pltpu.VMEM((ET,), jnp.float32),
            pltpu.VMEM((NPT,), jnp.float32),
            pltpu.VMEM((NPT,), jnp.float32),
            pltpu.VMEM((K,), jnp.int32),
            pltpu.VMEM((K,), jnp.int32),
            pltpu.VMEM((K,), jnp.int32),
            pltpu.VMEM((K,), jnp.int32),
            pltpu.VMEM((K, 128), jnp.float32),
            pltpu.VMEM((K, 128), jnp.float32),
            pltpu.SemaphoreType.DMA,
            pltpu.SemaphoreType.DMA,
        ],
    )
    def conv_kernel(h_hbm, esrc_hbm, edst_hbm, ew_hbm, out_hbm,
                    acc_sp, parts_sp, dis_sp,
                    deg_loc, dis_loc, esrc, edst, ew, enorm, degacc, tmp,
                    src_ch0, src_ch1, dst_ch0, dst_ch1, rows0, rows1,
                    sem0, sem1):
        c = lax.axis_index("c")
        s = lax.axis_index("s")
        z16 = jnp.zeros((L,), jnp.float32)

        def zero_rows(rr, _):
            for j in range(OUT // L):
                rows0[rr, pl.ds(j * L, L)] = z16
            return 0

        def zero_deg(i, _):
            deg_loc[pl.ds(i * L, L)] = z16
            return 0

        def deg_scatter(i, _):
            d16 = edst[pl.ds(i * L, L)]
            w16 = ew[pl.ds(i * L, L)]
            plsc.addupdate_scatter(deg_loc, [d16], w16)
            return 0

        def scale_chunk(rows_b, ci):
            def srow(rr, _):
                idx16 = jnp.zeros((L,), jnp.int32) + (ci * K + rr)
                nv = plsc.load_gather(enorm, [idx16])
                for j in range(OUT // L):
                    rows_b[rr, pl.ds(j * L, L)] = (
                        rows_b[rr, pl.ds(j * L, L)] * nv)
                return 0
            lax.fori_loop(0, K, srow, 0)

        def prep_fire(ci, src_b, dst_b, rows_b, sem):
            pltpu.sync_copy(esrc.at[pl.ds(ci * K, K)], src_b)
            pltpu.sync_copy(edst.at[pl.ds(ci * K, K)], dst_b)
            pltpu.async_copy(h_hbm.at[src_b], rows_b, sem)

        for r in range(RB):
            bi = c * RB + r

            lax.fori_loop(0, K, zero_rows, 0)
            lax.fori_loop(0, NP // L, zero_deg, 0)
            for k in range(NPT // K):
                pltpu.sync_copy(rows0,
                                acc_sp.at[pl.ds(s * NPT + k * K, K), :])
            plsc.subcore_barrier()

            off = bi * E2p + s * ET
            pltpu.sync_copy(esrc_hbm.at[pl.ds(off, ET)], esrc)
            pltpu.sync_copy(edst_hbm.at[pl.ds(off, ET)], edst)
            pltpu.sync_copy(ew_hbm.at[pl.ds(off, ET)], ew)

            lax.fori_loop(0, ET // L, deg_scatter, 0)
            pltpu.sync_copy(deg_loc, parts_sp.at[s])
            plsc.subcore_barrier()

            pltpu.sync_copy(parts_sp.at[0, pl.ds(s * NPT, NPT)], degacc)
            for p in range(1, NS):
                pltpu.sync_copy(parts_sp.at[p, pl.ds(s * NPT, NPT)], tmp)

                def accum(i, _):
                    degacc[pl.ds(i * L, L)] = (
                        degacc[pl.ds(i * L, L)] + tmp[pl.ds(i * L, L)])
                    return 0
                lax.fori_loop(0, NPT // L, accum, 0)

            def mk_dis(i, _):
                degacc[pl.ds(i * L, L)] = _rsqrt16(degacc[pl.ds(i * L, L)])
                return 0
            lax.fori_loop(0, NPT // L, mk_dis, 0)
            pltpu.sync_copy(degacc, dis_sp.at[pl.ds(s * NPT, NPT)])
            plsc.subcore_barrier()
            pltpu.sync_copy(dis_sp, dis_loc)

            bn = bi * N

            def norm_edges(i, _):
                sl = esrc[pl.ds(i * L, L)]
                dl = edst[pl.ds(i * L, L)]
                wv = ew[pl.ds(i * L, L)]
                ns = plsc.load_gather(dis_loc, [sl])
                nd = plsc.load_gather(dis_loc, [dl])
                enorm[pl.ds(i * L, L)] = ns * wv * nd
                esrc[pl.ds(i * L, L)] = sl + bn
                return 0
            lax.fori_loop(0, ET // L, norm_edges, 0)

            prep_fire(0, src_ch0, dst_ch0, rows0, sem0)

            def pair(gg, _):
                base = gg * 2
                prep_fire(base + 1, src_ch1, dst_ch1, rows1, sem1)
                pltpu.make_async_copy(h_hbm.at[src_ch0], rows0, sem0).wait()
                scale_chunk(rows0, base)
                pltpu.sync_copy(rows0, acc_sp.at[dst_ch0], add=True)

                @pl.when(base + 2 < C)
                def _():
                    prep_fire(base + 2, src_ch0, dst_ch0, rows0, sem0)
                pltpu.make_async_copy(h_hbm.at[src_ch1], rows1, sem1).wait()
                scale_chunk(rows1, base + 1)
                pltpu.sync_copy(rows1, acc_sp.at[dst_ch1], add=True)
                return 0
            lax.fori_loop(0, C // 2, pair, 0)
            plsc.subcore_barrier()

            @pl.when(s < n_full)
            def _():
                pltpu.sync_copy(acc_sp.at[pl.ds(s * NPT, NPT), :],
                                out_hbm.at[bi, pl.ds(s * NPT, NPT), :])
            if n_rem:
                @pl.when(s == n_full)
                def _():
                    pltpu.sync_copy(
                        acc_sp.at[pl.ds(n_full * NPT, n_rem), :],
                        out_hbm.at[bi, pl.ds(n_full * NPT, n_rem), :])
            plsc.subcore_barrier()

    return conv_kernel


def _stats_body(x_ref, o_ref):
    @pl.when(pl.program_id(0) == 0)
    def _():
        o_ref[...] = jnp.zeros_like(o_ref)
    xv = x_ref[...]
    o_ref[0:1, :] += jnp.sum(xv, axis=0, keepdims=True)
    o_ref[1:2, :] += jnp.sum(xv * xv, axis=0, keepdims=True)


def _tc_stats(flat):
    M, OUT = flat.shape
    BLK = 2500
    return pl.pallas_call(
        _stats_body,
        grid=(M // BLK,),
        in_specs=[pl.BlockSpec((BLK, OUT), lambda i: (i, 0))],
        out_specs=pl.BlockSpec((8, OUT), lambda i: (0, 0)),
        out_shape=jax.ShapeDtypeStruct((8, OUT), jnp.float32),
    )(flat)


def _bn_body(x_ref, a_ref, c_ref, o_ref):
    y = x_ref[...] * a_ref[...] + c_ref[...]
    o_ref[...] = jnp.where(y >= 0, y, 0.01 * y)


def _tc_bn(flat, a, cc):
    M, OUT = flat.shape
    BLK = 2500
    return pl.pallas_call(
        _bn_body,
        grid=(M // BLK,),
        in_specs=[pl.BlockSpec((BLK, OUT), lambda i: (i, 0)),
                  pl.BlockSpec((1, OUT), lambda i: (0, 0)),
                  pl.BlockSpec((1, OUT), lambda i: (0, 0))],
        out_specs=pl.BlockSpec((BLK, OUT), lambda i: (i, 0)),
        out_shape=jax.ShapeDtypeStruct((M, OUT), jnp.float32),
    )(flat, a.reshape(1, OUT), cc.reshape(1, OUT))


def kernel(x, edges, weights, W, b, bn_weight, bn_bias):
    B, N, D = x.shape
    OUT = W.shape[1]
    E = edges.shape[2]

    E2 = E + N
    E2p = ((E2 + NS * 128 - 1) // (NS * 128)) * (NS * 128)
    pad = E2p - E2
    loop = jnp.broadcast_to(jnp.arange(N, dtype=jnp.int32), (B, N))
    zpad_i = jnp.zeros((B, pad), jnp.int32)
    zpad_f = jnp.zeros((B, pad), jnp.float32)
    src2 = jnp.concatenate([edges[:, 0, :], loop, zpad_i], axis=1).reshape(-1)
    dst2 = jnp.concatenate([edges[:, 1, :], loop, zpad_i], axis=1).reshape(-1)
    ew2 = jnp.concatenate(
        [weights, jnp.ones((B, N), jnp.float32), zpad_f], axis=1).reshape(-1)

    h = _tc_matmul(x.reshape(B * N, D), W)

    conv = _make_sc_conv(B, N, OUT, E2p)(h, src2, dst2, ew2)

    flat = conv.reshape(B * N, OUT)
    st = _tc_stats(flat)
    M = B * N
    mean = st[0] / M
    var = st[1] / M - mean * mean
    inv = jax.lax.rsqrt(var + 1e-5)
    a = inv * bn_weight
    cc = bn_bias - mean * a
    out = _tc_bn(flat, a, cc)
    return out.reshape(B, N, OUT)


# speedup vs baseline: 19.5123x; 19.5123x over previous
import functools

import jax
import jax.numpy as jnp
from jax import lax
from jax.experimental import pallas as pl
from jax.experimental.pallas import tpu as pltpu
from jax.experimental.pallas import tpu_sc as plsc

NC = 2
NS = 16
L = 16


def _mm_body(x_ref, w_ref, o_ref):
    o_ref[...] = jnp.dot(x_ref[...], w_ref[...],
                         preferred_element_type=jnp.float32)


def _tc_matmul(xf, W):
    M, D = xf.shape
    OUT = W.shape[1]
    BLK = 2000
    return pl.pallas_call(
        _mm_body,
        grid=(M // BLK,),
        in_specs=[pl.BlockSpec((BLK, D), lambda i: (i, 0)),
                  pl.BlockSpec((D, OUT), lambda i: (0, 0))],
        out_specs=pl.BlockSpec((BLK, OUT), lambda i: (i, 0)),
        out_shape=jax.ShapeDtypeStruct((M, OUT), jnp.float32),
    )(xf, W)


def _rsqrt16(x):
    xi = plsc.bitcast(x, jnp.int32)
    yi = jnp.int32(0x5F3759DF) - lax.shift_right_logical(xi, 1)
    y = plsc.bitcast(yi, jnp.float32)
    hx = x * 0.5
    for _ in range(3):
        y = y * (1.5 - hx * y * y)
    return y


@functools.lru_cache(maxsize=None)
def _make_sc_conv(B, N, OUT, E2p):
    K = 128
    ROWS = E2p // K
    C = ROWS // NS
    RB = B // NC
    NPT = N // NS
    DR = -(-((N + K - 1) // K) // NS) * NS
    DRT = DR // NS

    mesh = plsc.VectorSubcoreMesh(core_axis_name="c", subcore_axis_name="s")

    @functools.partial(
        pl.kernel,
        out_type=(jax.ShapeDtypeStruct((B, N, OUT), jnp.float32),
                  jax.ShapeDtypeStruct((NC, NS, DR, K), jnp.float32)),
        mesh=mesh,
        scratch_types=[
            pltpu.VMEM_SHARED((N, OUT), jnp.float32),
            pltpu.VMEM_SHARED((DR, K), jnp.float32),
            pltpu.VMEM((DR, K), jnp.float32),
            pltpu.VMEM((DRT, K), jnp.float32),
            pltpu.VMEM((DRT, K), jnp.float32),
            pltpu.VMEM((DR,), jnp.int32),
            pltpu.VMEM((K,), jnp.int32),
            pltpu.VMEM((K,), jnp.int32),
            pltpu.VMEM((K,), jnp.int32),
            pltpu.VMEM((K,), jnp.int32),
            pltpu.VMEM((K,), jnp.float32),
            pltpu.VMEM((K,), jnp.float32),
            pltpu.VMEM((K,), jnp.int32),
            pltpu.VMEM((K,), jnp.int32),
            pltpu.VMEM((K,), jnp.int32),
            pltpu.VMEM((K,), jnp.int32),
            pltpu.VMEM((K,), jnp.float32),
            pltpu.VMEM((K,), jnp.float32),
            pltpu.VMEM((K, 128), jnp.float32),
            pltpu.VMEM((K, 128), jnp.float32),
            pltpu.SemaphoreType.DMA,
            pltpu.SemaphoreType.DMA,
            pltpu.SemaphoreType.DMA,
            pltpu.SemaphoreType.DMA,
        ],
        compiler_params=pltpu.CompilerParams(needs_layout_passes=False,
                                             use_tc_tiling_on_sc=False),
    )
    def conv_kernel(h_hbm, esrc_hbm, edst_hbm, ew_hbm, out_hbm, degp_hbm,
                    acc_sp, deg_sp, dis2d, sbuf, sbuf2, iota_r,
                    src_r0, src_r1, dst_r0, dst_r1, w_r0, w_r1,
                    srcg_h0, srcg_h1, dst_h0, dst_h1, norm_h0, norm_h1,
                    rows0, rows1, semE0, semE1, semG0, semG1):
        c = lax.axis_index("c")
        s = lax.axis_index("s")
        z16 = jnp.zeros((L,), jnp.float32)
        G = K // L

        def zero_rows0(rr, _):
            for j in range(OUT // L):
                rows0[rr, pl.ds(j * L, L)] = z16
            return 0

        def zero_dis2d(rr, _):
            for j in range(G):
                dis2d[rr, pl.ds(j * L, L)] = z16
            return 0

        def fire_dw(row, dst_r, w_r, sem):
            pltpu.async_copy(edst_hbm.at[row], dst_r, sem)
            pltpu.async_copy(ew_hbm.at[row], w_r, sem)

        def wait_dw(row, dst_r, w_r, sem):
            pltpu.make_async_copy(edst_hbm.at[row], dst_r, sem).wait()
            pltpu.make_async_copy(ew_hbm.at[row], w_r, sem).wait()

        def deg_scatter(dst_r, w_r):
            for j in range(G):
                d16 = dst_r[pl.ds(j * L, L)]
                w16 = w_r[pl.ds(j * L, L)]
                dr = lax.shift_right_logical(d16, 7)
                dc = jnp.bitwise_and(d16, 127)
                plsc.addupdate_scatter(dis2d, [dr, dc], w16)

        def fire_e(row, src_r, dst_r, w_r, sem):
            pltpu.async_copy(esrc_hbm.at[row], src_r, sem)
            pltpu.async_copy(edst_hbm.at[row], dst_r, sem)
            pltpu.async_copy(ew_hbm.at[row], w_r, sem)

        def wait_e(row, src_r, dst_r, w_r, sem):
            pltpu.make_async_copy(esrc_hbm.at[row], src_r, sem).wait()
            pltpu.make_async_copy(edst_hbm.at[row], dst_r, sem).wait()
            pltpu.make_async_copy(ew_hbm.at[row], w_r, sem).wait()

        def prep(bn, src_r, dst_r, w_r, srcg_h, dst_h, norm_h):
            for j in range(G):
                sl = src_r[pl.ds(j * L, L)]
                dl = dst_r[pl.ds(j * L, L)]
                wv = w_r[pl.ds(j * L, L)]
                ns = plsc.load_gather(
                    dis2d, [lax.shift_right_logical(sl, 7),
                            jnp.bitwise_and(sl, 127)])
                nd = plsc.load_gather(
                    dis2d, [lax.shift_right_logical(dl, 7),
                            jnp.bitwise_and(dl, 127)])
                norm_h[pl.ds(j * L, L)] = ns * wv * nd
                srcg_h[pl.ds(j * L, L)] = sl + bn
                dst_h[pl.ds(j * L, L)] = dl

        def fire_g(srcg_h, rows_b, sem):
            pltpu.async_copy(h_hbm.at[srcg_h], rows_b, sem)

        def wait_g(srcg_h, rows_b, sem):
            pltpu.make_async_copy(h_hbm.at[srcg_h], rows_b, sem).wait()

        def scale(rows_b, norm_h):
            def srow(rr, _):
                idx16 = jnp.zeros((L,), jnp.int32) + rr
                nv = plsc.load_gather(norm_h, [idx16])
                for j in range(OUT // L):
                    rows_b[rr, pl.ds(j * L, L)] = (
                        rows_b[rr, pl.ds(j * L, L)] * nv)
                return 0
            lax.fori_loop(0, K, srow, 0)

        for j in range(DR // L):
            iota_r[pl.ds(j * L, L)] = lax.iota(jnp.int32, L) + j * L

        for r in range(RB):
            bi = c * RB + r
            row0 = bi * (NS * C) + s * C

            lax.fori_loop(0, K, zero_rows0, 0)
            lax.fori_loop(0, DR, zero_dis2d, 0)
            for k in range(NPT // K):
                pltpu.sync_copy(rows0, acc_sp.at[pl.ds(s * NPT + k * K, K), :])
            rem = NPT - (NPT // K) * K
            if rem:
                pltpu.sync_copy(
                    rows0.at[pl.ds(0, rem), :],
                    acc_sp.at[pl.ds(s * NPT + (NPT // K) * K, rem), :])
            pltpu.sync_copy(rows0.at[pl.ds(0, DRT), :],
                            deg_sp.at[pl.ds(s * DRT, DRT), :])
            plsc.subcore_barrier()

            def dchunk(ci, _):
                row = row0 + ci
                pltpu.sync_copy(edst_hbm.at[row], dst_r0)
                pltpu.sync_copy(ew_hbm.at[row], w_r0)
                deg_scatter(dst_r0, w_r0)
                return 0
            lax.fori_loop(0, C, dchunk, 0)
            pltpu.sync_copy(dis2d, degp_hbm.at[c, s])
            plsc.subcore_barrier()

            pltpu.sync_copy(degp_hbm.at[c, 0, pl.ds(s * DRT, DRT), :], sbuf)
            for p in range(1, NS):
                pltpu.sync_copy(
                    degp_hbm.at[c, p, pl.ds(s * DRT, DRT), :], sbuf2)
                for rr in range(DRT):
                    for j in range(G):
                        sbuf[rr, pl.ds(j * L, L)] = (
                            sbuf[rr, pl.ds(j * L, L)]
                            + sbuf2[rr, pl.ds(j * L, L)])
            for rr in range(DRT):
                for j in range(G):
                    sbuf[rr, pl.ds(j * L, L)] = _rsqrt16(
                        sbuf[rr, pl.ds(j * L, L)])
            pltpu.sync_copy(sbuf, deg_sp.at[pl.ds(s * DRT, DRT), :])
            plsc.subcore_barrier()
            pltpu.sync_copy(deg_sp, dis2d)

            bn = bi * N

            def pchunk(ci, _):
                row = row0 + ci
                pltpu.sync_copy(esrc_hbm.at[row], src_r0)
                pltpu.sync_copy(edst_hbm.at[row], dst_r0)
                pltpu.sync_copy(ew_hbm.at[row], w_r0)
                prep(bn, src_r0, dst_r0, w_r0, srcg_h0, dst_h0, norm_h0)
                fire_g(srcg_h0, rows0, semG0)
                wait_g(srcg_h0, rows0, semG0)
                scale(rows0, norm_h0)
                pltpu.sync_copy(rows0, acc_sp.at[dst_h0], add=True)
                return 0
            lax.fori_loop(0, C, pchunk, 0)
            plsc.subcore_barrier()

            nko = (NPT + K - 1) // K
            for ko in range(nko):
                sz = K if (ko + 1) * K <= NPT else NPT - ko * K
                pltpu.sync_copy(
                    acc_sp.at[pl.ds(s * NPT + ko * K, sz), :],
                    rows0.at[pl.ds(0, sz), :])
                pltpu.sync_copy(
                    rows0.at[pl.ds(0, sz), :],
                    out_hbm.at[bi, pl.ds(s * NPT + ko * K, sz), :])
            plsc.subcore_barrier()

    return conv_kernel


@functools.lru_cache(maxsize=None)
def _make_sc_t1(B, N, OUT):
    NP = 10240
    NPT = NP // NS
    K = 128
    RB = B // NC
    mesh = plsc.VectorSubcoreMesh(core_axis_name="c", subcore_axis_name="s")

    @functools.partial(
        pl.kernel,
        out_type=jax.ShapeDtypeStruct((B, NP, OUT), jnp.float32),
        mesh=mesh,
        scratch_types=[
            pltpu.VMEM((K,), jnp.int32),
            pltpu.VMEM((K, 128), jnp.float32),
            pltpu.SemaphoreType.DMA,
        ],
        compiler_params=pltpu.CompilerParams(needs_layout_passes=False,
                                             use_tc_tiling_on_sc=False),
    )
    def t1_kernel(h_hbm, out_hbm, idx, rows, sem):
        c = lax.axis_index("c")
        s = lax.axis_index("s")
        for r in range(RB):
            bi = c * RB + r
            for k in range(NPT // K):
                base = bi * N + s * NPT + k * K
                for j in range(K // L):
                    idx[pl.ds(j * L, L)] = lax.iota(jnp.int32, L) + (
                        base + j * L)
                pltpu.async_copy(h_hbm.at[idx], rows, sem).wait()
                pltpu.sync_copy(
                    rows, out_hbm.at[bi, pl.ds(s * NPT + k * K, K), :])
            plsc.subcore_barrier()

    return t1_kernel


def _stats_body(x_ref, o_ref):
    @pl.when(pl.program_id(0) == 0)
    def _():
        o_ref[...] = jnp.zeros_like(o_ref)
    xv = x_ref[...]
    o_ref[0:1, :] += jnp.sum(xv, axis=0, keepdims=True)
    o_ref[1:2, :] += jnp.sum(xv * xv, axis=0, keepdims=True)


def _tc_stats(flat):
    M, OUT = flat.shape
    BLK = 2000
    return pl.pallas_call(
        _stats_body,
        grid=(M // BLK,),
        in_specs=[pl.BlockSpec((BLK, OUT), lambda i: (i, 0))],
        out_specs=pl.BlockSpec((8, OUT), lambda i: (0, 0)),
        out_shape=jax.ShapeDtypeStruct((8, OUT), jnp.float32),
    )(flat)


def _bn_body(x_ref, a_ref, c_ref, o_ref):
    y = x_ref[...] * a_ref[...] + c_ref[...]
    o_ref[...] = jnp.where(y >= 0, y, 0.01 * y)


def _tc_bn(flat, a, cc):
    M, OUT = flat.shape
    BLK = 2000
    return pl.pallas_call(
        _bn_body,
        grid=(M // BLK,),
        in_specs=[pl.BlockSpec((BLK, OUT), lambda i: (i, 0)),
                  pl.BlockSpec((1, OUT), lambda i: (0, 0)),
                  pl.BlockSpec((1, OUT), lambda i: (0, 0))],
        out_specs=pl.BlockSpec((BLK, OUT), lambda i: (i, 0)),
        out_shape=jax.ShapeDtypeStruct((M, OUT), jnp.float32),
    )(flat, a.reshape(1, OUT), cc.reshape(1, OUT))


def kernel(x, edges, weights, W, b, bn_weight, bn_bias):
    B, N, D = x.shape
    OUT = W.shape[1]
    E = edges.shape[2]

    E2 = E + N
    E2p = ((E2 + NS * 128 - 1) // (NS * 128)) * (NS * 128)
    pad = E2p - E2
    loop = jnp.broadcast_to(jnp.arange(N, dtype=jnp.int32), (B, N))
    zpad_i = jnp.zeros((B, pad), jnp.int32)
    zpad_f = jnp.zeros((B, pad), jnp.float32)
    src2 = jnp.concatenate([edges[:, 0, :], loop, zpad_i],
                           axis=1).reshape(-1, 128)
    dst2 = jnp.concatenate([edges[:, 1, :], loop, zpad_i],
                           axis=1).reshape(-1, 128)
    ew2 = jnp.concatenate(
        [weights, jnp.ones((B, N), jnp.float32), zpad_f],
        axis=1).reshape(-1, 128)

    h = _tc_matmul(x.reshape(B * N, D), W)

    conv, _ = _make_sc_conv(B, N, OUT, E2p)(h, src2, dst2, ew2)

    flat = conv.reshape(B * N, OUT)
    st = _tc_stats(flat)
    M = B * N
    mean = st[0] / M
    var = st[1] / M - mean * mean
    inv = jax.lax.rsqrt(var + 1e-5)
    a = inv * bn_weight
    cc = bn_bias - mean * a
    out = _tc_bn(flat, a, cc)
    return out.reshape(B, N, OUT)


# speedup vs baseline: 28.5083x; 1.4610x over previous
import functools

import jax
import jax.numpy as jnp
from jax import lax
from jax.experimental import pallas as pl
from jax.experimental.pallas import tpu as pltpu
from jax.experimental.pallas import tpu_sc as plsc

NC = 2
NS = 16
L = 16


def _mm_body(x_ref, w_ref, o_ref):
    o_ref[...] = jnp.dot(x_ref[...], w_ref[...],
                         preferred_element_type=jnp.float32)


def _tc_matmul(xf, W):
    M, D = xf.shape
    OUT = W.shape[1]
    BLK = 2000
    return pl.pallas_call(
        _mm_body,
        grid=(M // BLK,),
        in_specs=[pl.BlockSpec((BLK, D), lambda i: (i, 0)),
                  pl.BlockSpec((D, OUT), lambda i: (0, 0))],
        out_specs=pl.BlockSpec((BLK, OUT), lambda i: (i, 0)),
        out_shape=jax.ShapeDtypeStruct((M, OUT), jnp.float32),
    )(xf, W)


def _rsqrt16(x):
    xi = plsc.bitcast(x, jnp.int32)
    yi = jnp.int32(0x5F3759DF) - lax.shift_right_logical(xi, 1)
    y = plsc.bitcast(yi, jnp.float32)
    hx = x * 0.5
    for _ in range(3):
        y = y * (1.5 - hx * y * y)
    return y


@functools.lru_cache(maxsize=None)
def _make_sc_conv(B, N, OUT, E2p):
    K = 128
    ROWS = E2p // K
    C = ROWS // NS
    RB = B // NC
    NPT = N // NS
    DR = -(-((N + K - 1) // K) // NS) * NS
    DRT = DR // NS

    mesh = plsc.VectorSubcoreMesh(core_axis_name="c", subcore_axis_name="s")

    @functools.partial(
        pl.kernel,
        out_type=(jax.ShapeDtypeStruct((B, N, OUT), jnp.float32),
                  jax.ShapeDtypeStruct((NC, NS, DR, K), jnp.float32)),
        mesh=mesh,
        scratch_types=[
            pltpu.VMEM_SHARED((N, OUT), jnp.float32),
            pltpu.VMEM_SHARED((DR, K), jnp.float32),
            pltpu.VMEM((DR, K), jnp.float32),
            pltpu.VMEM((DRT, K), jnp.float32),
            pltpu.VMEM((DRT, K), jnp.float32),
            pltpu.VMEM((DR,), jnp.int32),
            pltpu.VMEM((K,), jnp.int32),
            pltpu.VMEM((K,), jnp.int32),
            pltpu.VMEM((K,), jnp.int32),
            pltpu.VMEM((K,), jnp.int32),
            pltpu.VMEM((K,), jnp.float32),
            pltpu.VMEM((K,), jnp.float32),
            pltpu.VMEM((K,), jnp.int32),
            pltpu.VMEM((K,), jnp.int32),
            pltpu.VMEM((K,), jnp.int32),
            pltpu.VMEM((K,), jnp.int32),
            pltpu.VMEM((K,), jnp.float32),
            pltpu.VMEM((K,), jnp.float32),
            pltpu.VMEM((K, 128), jnp.float32),
            pltpu.VMEM((K, 128), jnp.float32),
            pltpu.SemaphoreType.DMA,
            pltpu.SemaphoreType.DMA,
            pltpu.SemaphoreType.DMA,
            pltpu.SemaphoreType.DMA,
        ],
        compiler_params=pltpu.CompilerParams(needs_layout_passes=False,
                                             use_tc_tiling_on_sc=False),
    )
    def conv_kernel(h_hbm, esrc_hbm, edst_hbm, ew_hbm, out_hbm, degp_hbm,
                    acc_sp, deg_sp, dis2d, sbuf, sbuf2, iota_r,
                    src_r0, src_r1, dst_r0, dst_r1, w_r0, w_r1,
                    srcg_h0, srcg_h1, dst_h0, dst_h1, norm_h0, norm_h1,
                    rows0, rows1, semE0, semE1, semG0, semG1):
        c = lax.axis_index("c")
        s = lax.axis_index("s")
        z16 = jnp.zeros((L,), jnp.float32)
        G = K // L

        def zero_rows0(rr, _):
            for j in range(OUT // L):
                rows0[rr, pl.ds(j * L, L)] = z16
            return 0

        def zero_dis2d(rr, _):
            for j in range(G):
                dis2d[rr, pl.ds(j * L, L)] = z16
            return 0

        def fire_dw(row, dst_r, w_r, sem):
            pltpu.async_copy(edst_hbm.at[row], dst_r, sem)
            pltpu.async_copy(ew_hbm.at[row], w_r, sem)

        def wait_dw(row, dst_r, w_r, sem):
            pltpu.make_async_copy(edst_hbm.at[row], dst_r, sem).wait()
            pltpu.make_async_copy(ew_hbm.at[row], w_r, sem).wait()

        def deg_scatter(dst_r, w_r):
            for j in range(G):
                d16 = dst_r[pl.ds(j * L, L)]
                w16 = w_r[pl.ds(j * L, L)]
                dr = lax.shift_right_logical(d16, 7)
                dc = jnp.bitwise_and(d16, 127)
                plsc.addupdate_scatter(dis2d, [dr, dc], w16)

        def fire_e(row, src_r, dst_r, w_r, sem):
            pltpu.async_copy(esrc_hbm.at[row], src_r, sem)
            pltpu.async_copy(edst_hbm.at[row], dst_r, sem)
            pltpu.async_copy(ew_hbm.at[row], w_r, sem)

        def wait_e(row, src_r, dst_r, w_r, sem):
            pltpu.make_async_copy(esrc_hbm.at[row], src_r, sem).wait()
            pltpu.make_async_copy(edst_hbm.at[row], dst_r, sem).wait()
            pltpu.make_async_copy(ew_hbm.at[row], w_r, sem).wait()

        def prep(bn, src_r, dst_r, w_r, srcg_h, dst_h, norm_h):
            for j in range(G):
                sl = src_r[pl.ds(j * L, L)]
                dl = dst_r[pl.ds(j * L, L)]
                wv = w_r[pl.ds(j * L, L)]
                ns = plsc.load_gather(
                    dis2d, [lax.shift_right_logical(sl, 7),
                            jnp.bitwise_and(sl, 127)])
                nd = plsc.load_gather(
                    dis2d, [lax.shift_right_logical(dl, 7),
                            jnp.bitwise_and(dl, 127)])
                norm_h[pl.ds(j * L, L)] = ns * wv * nd
                srcg_h[pl.ds(j * L, L)] = sl + bn
                dst_h[pl.ds(j * L, L)] = dl

        def fire_g(srcg_h, rows_b, sem):
            pltpu.async_copy(h_hbm.at[srcg_h], rows_b, sem)

        def wait_g(srcg_h, rows_b, sem):
            pltpu.make_async_copy(h_hbm.at[srcg_h], rows_b, sem).wait()

        def scale(rows_b, norm_h):
            def srow(rr, _):
                idx16 = jnp.zeros((L,), jnp.int32) + rr
                nv = plsc.load_gather(norm_h, [idx16])
                for j in range(OUT // L):
                    rows_b[rr, pl.ds(j * L, L)] = (
                        rows_b[rr, pl.ds(j * L, L)] * nv)
                return 0
            lax.fori_loop(0, K, srow, 0)

        for j in range(DR // L):
            iota_r[pl.ds(j * L, L)] = lax.iota(jnp.int32, L) + j * L

        for r in range(RB):
            bi = c * RB + r
            row0 = bi * (NS * C) + s * C

            lax.fori_loop(0, K, zero_rows0, 0)
            lax.fori_loop(0, DR, zero_dis2d, 0)
            for k in range(NPT // K):
                pltpu.sync_copy(rows0, acc_sp.at[pl.ds(s * NPT + k * K, K), :])
            rem = NPT - (NPT // K) * K
            if rem:
                pltpu.sync_copy(
                    rows0.at[pl.ds(0, rem), :],
                    acc_sp.at[pl.ds(s * NPT + (NPT // K) * K, rem), :])
            pltpu.sync_copy(rows0.at[pl.ds(0, DRT), :],
                            deg_sp.at[pl.ds(s * DRT, DRT), :])
            plsc.subcore_barrier()

            fire_dw(row0, dst_r0, w_r0, semE0)

            def dpair(gg, _):
                b0 = row0 + gg * 2
                fire_dw(b0 + 1, dst_r1, w_r1, semE1)
                wait_dw(b0, dst_r0, w_r0, semE0)
                deg_scatter(dst_r0, w_r0)

                @pl.when(gg * 2 + 2 < C)
                def _():
                    fire_dw(b0 + 2, dst_r0, w_r0, semE0)
                wait_dw(b0 + 1, dst_r1, w_r1, semE1)
                deg_scatter(dst_r1, w_r1)
                return 0
            lax.fori_loop(0, C // 2, dpair, 0)
            pltpu.sync_copy(dis2d, degp_hbm.at[c, s])
            plsc.subcore_barrier()

            pltpu.sync_copy(degp_hbm.at[c, 0, pl.ds(s * DRT, DRT), :], sbuf)
            for p in range(1, NS):
                pltpu.sync_copy(
                    degp_hbm.at[c, p, pl.ds(s * DRT, DRT), :], sbuf2)
                for rr in range(DRT):
                    for j in range(G):
                        sbuf[rr, pl.ds(j * L, L)] = (
                            sbuf[rr, pl.ds(j * L, L)]
                            + sbuf2[rr, pl.ds(j * L, L)])
            for rr in range(DRT):
                for j in range(G):
                    sbuf[rr, pl.ds(j * L, L)] = _rsqrt16(
                        sbuf[rr, pl.ds(j * L, L)])
            pltpu.sync_copy(sbuf, deg_sp.at[pl.ds(s * DRT, DRT), :])
            plsc.subcore_barrier()
            pltpu.sync_copy(deg_sp, dis2d)

            bn = bi * N

            fire_e(row0, src_r0, dst_r0, w_r0, semE0)
            fire_e(row0 + 1, src_r1, dst_r1, w_r1, semE1)
            wait_e(row0, src_r0, dst_r0, w_r0, semE0)
            prep(bn, src_r0, dst_r0, w_r0, srcg_h0, dst_h0, norm_h0)
            fire_g(srcg_h0, rows0, semG0)

            def ppair(gg, _):
                b0 = row0 + gg * 2
                wait_e(b0 + 1, src_r1, dst_r1, w_r1, semE1)
                prep(bn, src_r1, dst_r1, w_r1, srcg_h1, dst_h1, norm_h1)
                fire_g(srcg_h1, rows1, semG1)

                @pl.when(gg * 2 + 2 < C)
                def _():
                    fire_e(b0 + 2, src_r0, dst_r0, w_r0, semE0)
                wait_g(srcg_h0, rows0, semG0)
                scale(rows0, norm_h0)
                pltpu.sync_copy(rows0, acc_sp.at[dst_h0], add=True)

                @pl.when(gg * 2 + 3 < C)
                def _():
                    fire_e(b0 + 3, src_r1, dst_r1, w_r1, semE1)
                wait_g(srcg_h1, rows1, semG1)
                scale(rows1, norm_h1)
                pltpu.sync_copy(rows1, acc_sp.at[dst_h1], add=True)

                @pl.when(gg * 2 + 2 < C)
                def _():
                    wait_e(b0 + 2, src_r0, dst_r0, w_r0, semE0)
                    prep(bn, src_r0, dst_r0, w_r0, srcg_h0, dst_h0, norm_h0)
                    fire_g(srcg_h0, rows0, semG0)
                return 0
            lax.fori_loop(0, C // 2, ppair, 0)
            plsc.subcore_barrier()

            nko = (NPT + K - 1) // K
            for ko in range(nko):
                sz = K if (ko + 1) * K <= NPT else NPT - ko * K
                pltpu.sync_copy(
                    acc_sp.at[pl.ds(s * NPT + ko * K, sz), :],
                    rows0.at[pl.ds(0, sz), :])
                pltpu.sync_copy(
                    rows0.at[pl.ds(0, sz), :],
                    out_hbm.at[bi, pl.ds(s * NPT + ko * K, sz), :])
            plsc.subcore_barrier()

    return conv_kernel


@functools.lru_cache(maxsize=None)
def _make_sc_t1(B, N, OUT):
    NP = 10240
    NPT = NP // NS
    K = 128
    RB = B // NC
    mesh = plsc.VectorSubcoreMesh(core_axis_name="c", subcore_axis_name="s")

    @functools.partial(
        pl.kernel,
        out_type=jax.ShapeDtypeStruct((B, NP, OUT), jnp.float32),
        mesh=mesh,
        scratch_types=[
            pltpu.VMEM((K,), jnp.int32),
            pltpu.VMEM((K, 128), jnp.float32),
            pltpu.SemaphoreType.DMA,
        ],
        compiler_params=pltpu.CompilerParams(needs_layout_passes=False,
                                             use_tc_tiling_on_sc=False),
    )
    def t1_kernel(h_hbm, out_hbm, idx, rows, sem):
        c = lax.axis_index("c")
        s = lax.axis_index("s")
        for r in range(RB):
            bi = c * RB + r
            for k in range(NPT // K):
                base = bi * N + s * NPT + k * K
                for j in range(K // L):
                    idx[pl.ds(j * L, L)] = lax.iota(jnp.int32, L) + (
                        base + j * L)
                pltpu.async_copy(h_hbm.at[idx], rows, sem).wait()
                pltpu.sync_copy(
                    rows, out_hbm.at[bi, pl.ds(s * NPT + k * K, K), :])
            plsc.subcore_barrier()

    return t1_kernel


def _stats_body(x_ref, o_ref):
    @pl.when(pl.program_id(0) == 0)
    def _():
        o_ref[...] = jnp.zeros_like(o_ref)
    xv = x_ref[...]
    o_ref[0:1, :] += jnp.sum(xv, axis=0, keepdims=True)
    o_ref[1:2, :] += jnp.sum(xv * xv, axis=0, keepdims=True)


def _tc_stats(flat):
    M, OUT = flat.shape
    BLK = 2000
    return pl.pallas_call(
        _stats_body,
        grid=(M // BLK,),
        in_specs=[pl.BlockSpec((BLK, OUT), lambda i: (i, 0))],
        out_specs=pl.BlockSpec((8, OUT), lambda i: (0, 0)),
        out_shape=jax.ShapeDtypeStruct((8, OUT), jnp.float32),
    )(flat)


def _bn_body(x_ref, a_ref, c_ref, o_ref):
    y = x_ref[...] * a_ref[...] + c_ref[...]
    o_ref[...] = jnp.where(y >= 0, y, 0.01 * y)


def _tc_bn(flat, a, cc):
    M, OUT = flat.shape
    BLK = 2000
    return pl.pallas_call(
        _bn_body,
        grid=(M // BLK,),
        in_specs=[pl.BlockSpec((BLK, OUT), lambda i: (i, 0)),
                  pl.BlockSpec((1, OUT), lambda i: (0, 0)),
                  pl.BlockSpec((1, OUT), lambda i: (0, 0))],
        out_specs=pl.BlockSpec((BLK, OUT), lambda i: (i, 0)),
        out_shape=jax.ShapeDtypeStruct((M, OUT), jnp.float32),
    )(flat, a.reshape(1, OUT), cc.reshape(1, OUT))


def kernel(x, edges, weights, W, b, bn_weight, bn_bias):
    B, N, D = x.shape
    OUT = W.shape[1]
    E = edges.shape[2]

    E2 = E + N
    E2p = ((E2 + NS * 128 - 1) // (NS * 128)) * (NS * 128)
    pad = E2p - E2
    loop = jnp.broadcast_to(jnp.arange(N, dtype=jnp.int32), (B, N))
    zpad_i = jnp.zeros((B, pad), jnp.int32)
    zpad_f = jnp.zeros((B, pad), jnp.float32)
    src2 = jnp.concatenate([edges[:, 0, :], loop, zpad_i],
                           axis=1).reshape(-1, 128)
    dst2 = jnp.concatenate([edges[:, 1, :], loop, zpad_i],
                           axis=1).reshape(-1, 128)
    ew2 = jnp.concatenate(
        [weights, jnp.ones((B, N), jnp.float32), zpad_f],
        axis=1).reshape(-1, 128)

    h = _tc_matmul(x.reshape(B * N, D), W)

    conv, _ = _make_sc_conv(B, N, OUT, E2p)(h, src2, dst2, ew2)

    flat = conv.reshape(B * N, OUT)
    st = _tc_stats(flat)
    M = B * N
    mean = st[0] / M
    var = st[1] / M - mean * mean
    inv = jax.lax.rsqrt(var + 1e-5)
    a = inv * bn_weight
    cc = bn_bias - mean * a
    out = _tc_bn(flat, a, cc)
    return out.reshape(B, N, OUT)


# speedup vs baseline: 30.3400x; 1.0642x over previous
import functools

import jax
import jax.numpy as jnp
from jax import lax
from jax.experimental import pallas as pl
from jax.experimental.pallas import tpu as pltpu
from jax.experimental.pallas import tpu_sc as plsc

NC = 2
NS = 16
L = 16


def _mm_body(x_ref, w_ref, o_ref):
    o_ref[...] = jnp.dot(x_ref[...], w_ref[...],
                         preferred_element_type=jnp.float32)


def _tc_matmul(xf, W):
    M, D = xf.shape
    OUT = W.shape[1]
    BLK = 2000
    return pl.pallas_call(
        _mm_body,
        grid=(M // BLK,),
        in_specs=[pl.BlockSpec((BLK, D), lambda i: (i, 0)),
                  pl.BlockSpec((D, OUT), lambda i: (0, 0))],
        out_specs=pl.BlockSpec((BLK, OUT), lambda i: (i, 0)),
        out_shape=jax.ShapeDtypeStruct((M, OUT), jnp.float32),
    )(xf, W)


def _rsqrt16(x):
    xi = plsc.bitcast(x, jnp.int32)
    yi = jnp.int32(0x5F3759DF) - lax.shift_right_logical(xi, 1)
    y = plsc.bitcast(yi, jnp.float32)
    hx = x * 0.5
    for _ in range(3):
        y = y * (1.5 - hx * y * y)
    return y


@functools.lru_cache(maxsize=None)
def _make_sc_conv(B, N, OUT, E2p):
    K = 128
    ROWS = E2p // K
    C = ROWS // NS
    RB = B // NC
    NPT = N // NS
    DR = -(-((N + K - 1) // K) // NS) * NS
    DRT = DR // NS

    mesh = plsc.VectorSubcoreMesh(core_axis_name="c", subcore_axis_name="s")

    @functools.partial(
        pl.kernel,
        out_type=(jax.ShapeDtypeStruct((B, N, OUT), jnp.float32),
                  jax.ShapeDtypeStruct((NC, NS, DR, K), jnp.float32)),
        mesh=mesh,
        scratch_types=[
            pltpu.VMEM_SHARED((N, OUT), jnp.float32),
            pltpu.VMEM_SHARED((DR, K), jnp.float32),
            pltpu.VMEM((DR, K), jnp.float32),
            pltpu.VMEM((DRT, K), jnp.float32),
            pltpu.VMEM((DRT, K), jnp.float32),
            pltpu.VMEM((DR,), jnp.int32),
            pltpu.VMEM((K,), jnp.int32),
            pltpu.VMEM((K,), jnp.int32),
            pltpu.VMEM((K,), jnp.int32),
            pltpu.VMEM((K,), jnp.int32),
            pltpu.VMEM((K,), jnp.float32),
            pltpu.VMEM((K,), jnp.float32),
            pltpu.VMEM((K,), jnp.int32),
            pltpu.VMEM((K,), jnp.int32),
            pltpu.VMEM((K,), jnp.int32),
            pltpu.VMEM((K,), jnp.int32),
            pltpu.VMEM((K,), jnp.float32),
            pltpu.VMEM((K,), jnp.float32),
            pltpu.VMEM((K, 128), jnp.float32),
            pltpu.VMEM((K, 128), jnp.float32),
            pltpu.SemaphoreType.DMA,
            pltpu.SemaphoreType.DMA,
            pltpu.SemaphoreType.DMA,
            pltpu.SemaphoreType.DMA,
            pltpu.SemaphoreType.DMA,
        ],
        compiler_params=pltpu.CompilerParams(needs_layout_passes=False,
                                             use_tc_tiling_on_sc=False),
    )
    def conv_kernel(h_hbm, esrc_hbm, edst_hbm, ew_hbm, out_hbm, degp_hbm,
                    acc_sp, deg_sp, dis2d, sbuf, sbuf2, iota_r,
                    src_r0, src_r1, dst_r0, dst_r1, w_r0, w_r1,
                    srcg_h0, srcg_h1, dst_h0, dst_h1, norm_h0, norm_h1,
                    rows0, rows1, semE0, semE1, semG0, semG1, semS0):
        c = lax.axis_index("c")
        s = lax.axis_index("s")
        z16 = jnp.zeros((L,), jnp.float32)
        G = K // L

        def zero_rows0(rr, _):
            for j in range(OUT // L):
                rows0[rr, pl.ds(j * L, L)] = z16
            return 0

        def zero_dis2d(rr, _):
            for j in range(G):
                dis2d[rr, pl.ds(j * L, L)] = z16
            return 0

        def fire_dw(row, dst_r, w_r, sem):
            pltpu.async_copy(edst_hbm.at[row], dst_r, sem)
            pltpu.async_copy(ew_hbm.at[row], w_r, sem)

        def wait_dw(row, dst_r, w_r, sem):
            pltpu.make_async_copy(edst_hbm.at[row], dst_r, sem).wait()
            pltpu.make_async_copy(ew_hbm.at[row], w_r, sem).wait()

        def deg_scatter(dst_r, w_r):
            for j in range(G):
                d16 = dst_r[pl.ds(j * L, L)]
                w16 = w_r[pl.ds(j * L, L)]
                dr = lax.shift_right_logical(d16, 7)
                dc = jnp.bitwise_and(d16, 127)
                plsc.addupdate_scatter(dis2d, [dr, dc], w16)

        def fire_e(row, src_r, dst_r, w_r, sem):
            pltpu.async_copy(esrc_hbm.at[row], src_r, sem)
            pltpu.async_copy(edst_hbm.at[row], dst_r, sem)
            pltpu.async_copy(ew_hbm.at[row], w_r, sem)

        def wait_e(row, src_r, dst_r, w_r, sem):
            pltpu.make_async_copy(esrc_hbm.at[row], src_r, sem).wait()
            pltpu.make_async_copy(edst_hbm.at[row], dst_r, sem).wait()
            pltpu.make_async_copy(ew_hbm.at[row], w_r, sem).wait()

        def prep(bn, src_r, dst_r, w_r, srcg_h, dst_h, norm_h):
            for j in range(G):
                sl = src_r[pl.ds(j * L, L)]
                dl = dst_r[pl.ds(j * L, L)]
                wv = w_r[pl.ds(j * L, L)]
                ns = plsc.load_gather(
                    dis2d, [lax.shift_right_logical(sl, 7),
                            jnp.bitwise_and(sl, 127)])
                nd = plsc.load_gather(
                    dis2d, [lax.shift_right_logical(dl, 7),
                            jnp.bitwise_and(dl, 127)])
                norm_h[pl.ds(j * L, L)] = ns * wv * nd
                srcg_h[pl.ds(j * L, L)] = sl + bn
                dst_h[pl.ds(j * L, L)] = dl

        def fire_g(srcg_h, rows_b, sem):
            pltpu.async_copy(h_hbm.at[srcg_h], rows_b, sem)

        def wait_g(srcg_h, rows_b, sem):
            pltpu.make_async_copy(h_hbm.at[srcg_h], rows_b, sem).wait()

        def scale(rows_b, norm_h):
            def srow(rr, _):
                idx16 = jnp.zeros((L,), jnp.int32) + rr
                nv = plsc.load_gather(norm_h, [idx16])
                for j in range(OUT // L):
                    rows_b[rr, pl.ds(j * L, L)] = (
                        rows_b[rr, pl.ds(j * L, L)] * nv)
                return 0
            lax.fori_loop(0, K, srow, 0)

        for j in range(DR // L):
            iota_r[pl.ds(j * L, L)] = lax.iota(jnp.int32, L) + j * L

        for r in range(RB):
            bi = c * RB + r
            row0 = bi * (NS * C) + s * C

            lax.fori_loop(0, K, zero_rows0, 0)
            lax.fori_loop(0, DR, zero_dis2d, 0)
            for k in range(NPT // K):
                pltpu.sync_copy(rows0, acc_sp.at[pl.ds(s * NPT + k * K, K), :])
            rem = NPT - (NPT // K) * K
            if rem:
                pltpu.sync_copy(
                    rows0.at[pl.ds(0, rem), :],
                    acc_sp.at[pl.ds(s * NPT + (NPT // K) * K, rem), :])
            pltpu.sync_copy(rows0.at[pl.ds(0, DRT), :],
                            deg_sp.at[pl.ds(s * DRT, DRT), :])
            plsc.subcore_barrier()

            fire_dw(row0, dst_r0, w_r0, semE0)

            def dpair(gg, _):
                b0 = row0 + gg * 2
                fire_dw(b0 + 1, dst_r1, w_r1, semE1)
                wait_dw(b0, dst_r0, w_r0, semE0)
                deg_scatter(dst_r0, w_r0)

                @pl.when(gg * 2 + 2 < C)
                def _():
                    fire_dw(b0 + 2, dst_r0, w_r0, semE0)
                wait_dw(b0 + 1, dst_r1, w_r1, semE1)
                deg_scatter(dst_r1, w_r1)
                return 0
            lax.fori_loop(0, C // 2, dpair, 0)
            pltpu.sync_copy(dis2d, degp_hbm.at[c, s])
            plsc.subcore_barrier()

            pltpu.sync_copy(degp_hbm.at[c, 0, pl.ds(s * DRT, DRT), :], sbuf)
            for p in range(1, NS):
                pltpu.sync_copy(
                    degp_hbm.at[c, p, pl.ds(s * DRT, DRT), :], sbuf2)
                for rr in range(DRT):
                    for j in range(G):
                        sbuf[rr, pl.ds(j * L, L)] = (
                            sbuf[rr, pl.ds(j * L, L)]
                            + sbuf2[rr, pl.ds(j * L, L)])
            for rr in range(DRT):
                for j in range(G):
                    sbuf[rr, pl.ds(j * L, L)] = _rsqrt16(
                        sbuf[rr, pl.ds(j * L, L)])
            pltpu.sync_copy(sbuf, deg_sp.at[pl.ds(s * DRT, DRT), :])
            plsc.subcore_barrier()
            pltpu.sync_copy(deg_sp, dis2d)

            bn = bi * N

            fire_e(row0, src_r0, dst_r0, w_r0, semE0)
            fire_e(row0 + 1, src_r1, dst_r1, w_r1, semE1)
            wait_e(row0, src_r0, dst_r0, w_r0, semE0)
            prep(bn, src_r0, dst_r0, w_r0, srcg_h0, dst_h0, norm_h0)
            fire_g(srcg_h0, rows0, semG0)

            def ppair(gg, _):
                b0 = row0 + gg * 2
                wait_e(b0 + 1, src_r1, dst_r1, w_r1, semE1)
                prep(bn, src_r1, dst_r1, w_r1, srcg_h1, dst_h1, norm_h1)
                fire_g(srcg_h1, rows1, semG1)

                @pl.when(gg * 2 + 2 < C)
                def _():
                    fire_e(b0 + 2, src_r0, dst_r0, w_r0, semE0)
                wait_g(srcg_h0, rows0, semG0)
                scale(rows0, norm_h0)
                pltpu.async_copy(rows0, acc_sp.at[dst_h0], semS0, add=True)

                @pl.when(gg * 2 + 3 < C)
                def _():
                    fire_e(b0 + 3, src_r1, dst_r1, w_r1, semE1)
                wait_g(srcg_h1, rows1, semG1)
                scale(rows1, norm_h1)
                pltpu.sync_copy(rows1, acc_sp.at[dst_h1], add=True)

                @pl.when(gg * 2 + 2 < C)
                def _():
                    wait_e(b0 + 2, src_r0, dst_r0, w_r0, semE0)
                    pltpu.make_async_copy(rows0, acc_sp.at[dst_h0],
                                          semS0).wait()
                    prep(bn, src_r0, dst_r0, w_r0, srcg_h0, dst_h0, norm_h0)
                    fire_g(srcg_h0, rows0, semG0)
                return 0
            lax.fori_loop(0, C // 2, ppair, 0)
            pltpu.make_async_copy(rows0, acc_sp.at[dst_h0], semS0).wait()
            plsc.subcore_barrier()

            nko = (NPT + K - 1) // K
            for ko in range(nko):
                sz = K if (ko + 1) * K <= NPT else NPT - ko * K
                pltpu.sync_copy(
                    acc_sp.at[pl.ds(s * NPT + ko * K, sz), :],
                    rows0.at[pl.ds(0, sz), :])
                pltpu.sync_copy(
                    rows0.at[pl.ds(0, sz), :],
                    out_hbm.at[bi, pl.ds(s * NPT + ko * K, sz), :])
            plsc.subcore_barrier()

    return conv_kernel


@functools.lru_cache(maxsize=None)
def _make_sc_t1(B, N, OUT):
    NP = 10240
    NPT = NP // NS
    K = 128
    RB = B // NC
    mesh = plsc.VectorSubcoreMesh(core_axis_name="c", subcore_axis_name="s")

    @functools.partial(
        pl.kernel,
        out_type=jax.ShapeDtypeStruct((B, NP, OUT), jnp.float32),
        mesh=mesh,
        scratch_types=[
            pltpu.VMEM((K,), jnp.int32),
            pltpu.VMEM((K, 128), jnp.float32),
            pltpu.SemaphoreType.DMA,
        ],
        compiler_params=pltpu.CompilerParams(needs_layout_passes=False,
                                             use_tc_tiling_on_sc=False),
    )
    def t1_kernel(h_hbm, out_hbm, idx, rows, sem):
        c = lax.axis_index("c")
        s = lax.axis_index("s")
        for r in range(RB):
            bi = c * RB + r
            for k in range(NPT // K):
                base = bi * N + s * NPT + k * K
                for j in range(K // L):
                    idx[pl.ds(j * L, L)] = lax.iota(jnp.int32, L) + (
                        base + j * L)
                pltpu.async_copy(h_hbm.at[idx], rows, sem).wait()
                pltpu.sync_copy(
                    rows, out_hbm.at[bi, pl.ds(s * NPT + k * K, K), :])
            plsc.subcore_barrier()

    return t1_kernel


def _stats_body(x_ref, o_ref):
    @pl.when(pl.program_id(0) == 0)
    def _():
        o_ref[...] = jnp.zeros_like(o_ref)
    xv = x_ref[...]
    o_ref[0:1, :] += jnp.sum(xv, axis=0, keepdims=True)
    o_ref[1:2, :] += jnp.sum(xv * xv, axis=0, keepdims=True)


def _tc_stats(flat):
    M, OUT = flat.shape
    BLK = 2000
    return pl.pallas_call(
        _stats_body,
        grid=(M // BLK,),
        in_specs=[pl.BlockSpec((BLK, OUT), lambda i: (i, 0))],
        out_specs=pl.BlockSpec((8, OUT), lambda i: (0, 0)),
        out_shape=jax.ShapeDtypeStruct((8, OUT), jnp.float32),
    )(flat)


def _bn_body(x_ref, a_ref, c_ref, o_ref):
    y = x_ref[...] * a_ref[...] + c_ref[...]
    o_ref[...] = jnp.where(y >= 0, y, 0.01 * y)


def _tc_bn(flat, a, cc):
    M, OUT = flat.shape
    BLK = 2000
    return pl.pallas_call(
        _bn_body,
        grid=(M // BLK,),
        in_specs=[pl.BlockSpec((BLK, OUT), lambda i: (i, 0)),
                  pl.BlockSpec((1, OUT), lambda i: (0, 0)),
                  pl.BlockSpec((1, OUT), lambda i: (0, 0))],
        out_specs=pl.BlockSpec((BLK, OUT), lambda i: (i, 0)),
        out_shape=jax.ShapeDtypeStruct((M, OUT), jnp.float32),
    )(flat, a.reshape(1, OUT), cc.reshape(1, OUT))


def kernel(x, edges, weights, W, b, bn_weight, bn_bias):
    B, N, D = x.shape
    OUT = W.shape[1]
    E = edges.shape[2]

    E2 = E + N
    E2p = ((E2 + NS * 128 - 1) // (NS * 128)) * (NS * 128)
    pad = E2p - E2
    loop = jnp.broadcast_to(jnp.arange(N, dtype=jnp.int32), (B, N))
    zpad_i = jnp.zeros((B, pad), jnp.int32)
    zpad_f = jnp.zeros((B, pad), jnp.float32)
    src2 = jnp.concatenate([edges[:, 0, :], loop, zpad_i],
                           axis=1).reshape(-1, 128)
    dst2 = jnp.concatenate([edges[:, 1, :], loop, zpad_i],
                           axis=1).reshape(-1, 128)
    ew2 = jnp.concatenate(
        [weights, jnp.ones((B, N), jnp.float32), zpad_f],
        axis=1).reshape(-1, 128)

    h = _tc_matmul(x.reshape(B * N, D), W)

    conv, _ = _make_sc_conv(B, N, OUT, E2p)(h, src2, dst2, ew2)

    flat = conv.reshape(B * N, OUT)
    st = _tc_stats(flat)
    M = B * N
    mean = st[0] / M
    var = st[1] / M - mean * mean
    inv = jax.lax.rsqrt(var + 1e-5)
    a = inv * bn_weight
    cc = bn_bias - mean * a
    out = _tc_bn(flat, a, cc)
    return out.reshape(B, N, OUT)
